# baseline (device time: 258015 ns/iter reference)
import jax
import jax.numpy as jnp
from jax import lax
from jax.experimental import pallas as pl
from jax.experimental.pallas import tpu as pltpu

N_DEV = 32
SQ = 1024
DM = 1024
CHUNK = DM // N_DEV
DH = 128


def _allreduce_body(p_ref, out_ref, comm_ref,
                    send_sem_rs, recv_sems_rs, send_sem_ag, recv_sems_ag):
    my = lax.axis_index("i")
    left = lax.rem(my + N_DEV - 1, N_DEV)
    right = lax.rem(my + 1, N_DEV)

    barrier_sem = pltpu.get_barrier_semaphore()
    for nbr in (left, right):
        pl.semaphore_signal(
            barrier_sem, inc=1,
            device_id=(nbr,), device_id_type=pl.DeviceIdType.MESH,
        )
    pl.semaphore_wait(barrier_sem, 2)

    out_ref[...] = p_ref[...]

    for s in range(N_DEV - 1):
        c_send = lax.rem(my + (2 * N_DEV - s), N_DEV)
        c_recv = lax.rem(my + (2 * N_DEV - s - 1), N_DEV)
        rdma = pltpu.make_async_remote_copy(
            src_ref=out_ref.at[pl.ds(c_send * CHUNK, CHUNK), :],
            dst_ref=comm_ref.at[s],
            send_sem=send_sem_rs,
            recv_sem=recv_sems_rs.at[s],
            device_id=(right,),
            device_id_type=pl.DeviceIdType.MESH,
        )
        rdma.start()
        rdma.wait()
        rows = pl.ds(c_recv * CHUNK, CHUNK)
        out_ref[rows, :] = out_ref[rows, :] + comm_ref[s]

    for t in range(N_DEV - 1):
        c_fwd = lax.rem(my + (2 * N_DEV + 1 - t), N_DEV)
        rows = pl.ds(c_fwd * CHUNK, CHUNK)
        rdma = pltpu.make_async_remote_copy(
            src_ref=out_ref.at[rows, :],
            dst_ref=out_ref.at[rows, :],
            send_sem=send_sem_ag,
            recv_sem=recv_sems_ag.at[t],
            device_id=(right,),
            device_id_type=pl.DeviceIdType.MESH,
        )
        rdma.start()
        rdma.wait()


def _ring_allreduce(partial):
    return pl.pallas_call(
        _allreduce_body,
        out_shape=jax.ShapeDtypeStruct((DM, DM), jnp.float32),
        in_specs=[pl.BlockSpec(memory_space=pltpu.VMEM)],
        out_specs=pl.BlockSpec(memory_space=pltpu.VMEM),
        scratch_shapes=[
            pltpu.VMEM((N_DEV - 1, CHUNK, DM), jnp.float32),
            pltpu.SemaphoreType.DMA,
            pltpu.SemaphoreType.DMA((N_DEV - 1,)),
            pltpu.SemaphoreType.DMA,
            pltpu.SemaphoreType.DMA((N_DEV - 1,)),
        ],
        compiler_params=pltpu.CompilerParams(collective_id=0),
    )(partial)


def kernel(x, Wq, K_ext, V_ext, Wo):
    i = lax.axis_index("i")
    hs = Wq.shape[1] // DH

    xb = x[0].astype(jnp.bfloat16)
    q = jnp.dot(xb, Wq.astype(jnp.bfloat16),
                preferred_element_type=jnp.float32)
    q = q.reshape(SQ, hs, DH).astype(jnp.bfloat16)

    k = lax.dynamic_slice_in_dim(K_ext[0], i * hs, hs, axis=1)
    v = lax.dynamic_slice_in_dim(V_ext[0], i * hs, hs, axis=1)
    k = k.astype(jnp.bfloat16)
    v = v.astype(jnp.bfloat16)

    scores = jnp.einsum("ihd,jhd->hij", q, k,
                        preferred_element_type=jnp.float32)
    scores = scores * 0.08838834764831843

    qi = jnp.arange(SQ)[:, None]
    ki = jnp.arange(SQ)[None, :]
    mask = (jnp.abs(qi - ki) <= 128) | (ki < 32) | (qi < 32)
    scores = jnp.where(mask[None], scores, -1e9)

    m = scores.max(axis=-1, keepdims=True)
    w = jnp.exp(scores - m)
    w = w / w.sum(axis=-1, keepdims=True)

    ctx = jnp.einsum("hij,jhd->ihd", w.astype(jnp.bfloat16), v,
                     preferred_element_type=jnp.float32)
    ctx = ctx.reshape(SQ, hs * DH).astype(jnp.bfloat16)

    partial = jnp.dot(ctx, Wo.astype(jnp.bfloat16),
                      preferred_element_type=jnp.float32)

    out = _ring_allreduce(partial)
    return out.reshape(1, SQ, DM)


# device time: 173670 ns/iter; 1.4857x vs baseline; 1.4857x over previous
import jax
import jax.numpy as jnp
from jax import lax
from jax.experimental import pallas as pl
from jax.experimental.pallas import tpu as pltpu

N_DEV = 32
PLANE = 8
NZ = 4
SQ = 1024
DM = 1024
CHUNK = DM // PLANE
SUB = CHUNK // NZ
DH = 128

_MESH = pl.DeviceIdType.MESH


def _allreduce_body(p_ref, out_ref, comm1, zcomm,
                    send_p1, recv_p1, send_b1, recv_b1,
                    send_b2, recv_b2, send_p3, recv_p3, p1done):
    my = lax.axis_index("i")
    q = lax.rem(my, PLANE)
    zi = my // PLANE
    base = my - q
    pright = base + lax.rem(q + 1, PLANE)
    pleft = base + lax.rem(q + PLANE - 1, PLANE)

    barrier_sem = pltpu.get_barrier_semaphore()
    peers = [pleft, pright] + [
        lax.rem(zi + d, NZ) * PLANE + q for d in range(1, NZ)
    ]
    for pr in peers:
        pl.semaphore_signal(barrier_sem, inc=1,
                            device_id=(pr,), device_id_type=_MESH)
    pl.semaphore_wait(barrier_sem, len(peers))

    out_ref[...] = p_ref[...]

    for s in range(PLANE - 1):
        c_send = lax.rem(q + PLANE - s, PLANE)
        c_recv = lax.rem(q + PLANE - 1 - s, PLANE)
        rdma = pltpu.make_async_remote_copy(
            src_ref=out_ref.at[pl.ds(c_send * CHUNK, CHUNK), :],
            dst_ref=comm1.at[s],
            send_sem=send_p1,
            recv_sem=recv_p1.at[s],
            device_id=(pright,),
            device_id_type=_MESH,
        )
        rdma.start()
        rdma.wait()
        rows = pl.ds(c_recv * CHUNK, CHUNK)
        out_ref[rows, :] = out_ref[rows, :] + comm1[s]

    pl.semaphore_signal(p1done, inc=1, device_id=(pleft,),
                        device_id_type=_MESH)

    c_own = lax.rem(q + 1, PLANE)
    slab = c_own * CHUNK

    b1 = []
    for d in range(1, NZ):
        tz = lax.rem(zi + d, NZ)
        tgt = tz * PLANE + q
        rdma = pltpu.make_async_remote_copy(
            src_ref=out_ref.at[pl.ds(slab + tz * SUB, SUB), :],
            dst_ref=zcomm.at[zi],
            send_sem=send_b1.at[d - 1],
            recv_sem=recv_b1.at[zi],
            device_id=(tgt,),
            device_id_type=_MESH,
        )
        rdma.start()
        b1.append(rdma)
    for d in range(1, NZ):
        sz = lax.rem(zi + d, NZ)
        rw = pltpu.make_async_remote_copy(
            src_ref=zcomm.at[sz],
            dst_ref=zcomm.at[sz],
            send_sem=send_b1.at[d - 1],
            recv_sem=recv_b1.at[sz],
            device_id=(my,),
            device_id_type=_MESH,
        )
        rw.wait_recv()
    myrows = pl.ds(slab + zi * SUB, SUB)
    acc = out_ref[myrows, :]
    for d in range(1, NZ):
        sz = lax.rem(zi + d, NZ)
        acc = acc + zcomm[sz]
    out_ref[myrows, :] = acc
    for rdma in b1:
        rdma.wait_send()

    b2 = []
    for d in range(1, NZ):
        tz = lax.rem(zi + d, NZ)
        tgt = tz * PLANE + q
        rdma = pltpu.make_async_remote_copy(
            src_ref=out_ref.at[myrows, :],
            dst_ref=out_ref.at[myrows, :],
            send_sem=send_b2.at[d - 1],
            recv_sem=recv_b2.at[zi],
            device_id=(tgt,),
            device_id_type=_MESH,
        )
        rdma.start()
        b2.append(rdma)
    for d in range(1, NZ):
        sz = lax.rem(zi + d, NZ)
        rw = pltpu.make_async_remote_copy(
            src_ref=out_ref.at[pl.ds(slab + sz * SUB, SUB), :],
            dst_ref=out_ref.at[pl.ds(slab + sz * SUB, SUB), :],
            send_sem=send_b2.at[d - 1],
            recv_sem=recv_b2.at[sz],
            device_id=(my,),
            device_id_type=_MESH,
        )
        rw.wait_recv()
    for rdma in b2:
        rdma.wait_send()

    pl.semaphore_wait(p1done, 1)
    for t in range(PLANE - 1):
        c_fwd = lax.rem(q + PLANE + 1 - t, PLANE)
        rows = pl.ds(c_fwd * CHUNK, CHUNK)
        rdma = pltpu.make_async_remote_copy(
            src_ref=out_ref.at[rows, :],
            dst_ref=out_ref.at[rows, :],
            send_sem=send_p3,
            recv_sem=recv_p3.at[t],
            device_id=(pright,),
            device_id_type=_MESH,
        )
        rdma.start()
        rdma.wait()


def _ring_allreduce(partial):
    return pl.pallas_call(
        _allreduce_body,
        out_shape=jax.ShapeDtypeStruct((DM, DM), jnp.float32),
        in_specs=[pl.BlockSpec(memory_space=pltpu.VMEM)],
        out_specs=pl.BlockSpec(memory_space=pltpu.VMEM),
        scratch_shapes=[
            pltpu.VMEM((PLANE - 1, CHUNK, DM), jnp.float32),
            pltpu.VMEM((NZ, SUB, DM), jnp.float32),
            pltpu.SemaphoreType.DMA,
            pltpu.SemaphoreType.DMA((PLANE - 1,)),
            pltpu.SemaphoreType.DMA((NZ - 1,)),
            pltpu.SemaphoreType.DMA((NZ,)),
            pltpu.SemaphoreType.DMA((NZ - 1,)),
            pltpu.SemaphoreType.DMA((NZ,)),
            pltpu.SemaphoreType.DMA,
            pltpu.SemaphoreType.DMA((PLANE - 1,)),
            pltpu.SemaphoreType.REGULAR,
        ],
        compiler_params=pltpu.CompilerParams(collective_id=0),
    )(partial)


def kernel(x, Wq, K_ext, V_ext, Wo):
    i = lax.axis_index("i")
    hs = Wq.shape[1] // DH

    xb = x[0].astype(jnp.bfloat16)
    q = jnp.dot(xb, Wq.astype(jnp.bfloat16),
                preferred_element_type=jnp.float32)
    q = q.reshape(SQ, hs, DH).astype(jnp.bfloat16)

    k = lax.dynamic_slice_in_dim(K_ext[0], i * hs, hs, axis=1)
    v = lax.dynamic_slice_in_dim(V_ext[0], i * hs, hs, axis=1)
    k = k.astype(jnp.bfloat16)
    v = v.astype(jnp.bfloat16)

    scores = jnp.einsum("ihd,jhd->hij", q, k,
                        preferred_element_type=jnp.float32)
    scores = scores * 0.08838834764831843

    qi = jnp.arange(SQ)[:, None]
    ki = jnp.arange(SQ)[None, :]
    mask = (jnp.abs(qi - ki) <= 128) | (ki < 32) | (qi < 32)
    scores = jnp.where(mask[None], scores, -1e9)

    m = scores.max(axis=-1, keepdims=True)
    w = jnp.exp(scores - m)
    w = w / w.sum(axis=-1, keepdims=True)

    ctx = jnp.einsum("hij,jhd->ihd", w.astype(jnp.bfloat16), v,
                     preferred_element_type=jnp.float32)
    ctx = ctx.reshape(SQ, hs * DH).astype(jnp.bfloat16)

    partial = jnp.dot(ctx, Wo.astype(jnp.bfloat16),
                      preferred_element_type=jnp.float32)

    out = _ring_allreduce(partial)
    return out.reshape(1, SQ, DM)


# device time: 166714 ns/iter; 1.5477x vs baseline; 1.0417x over previous
import jax
import jax.numpy as jnp
from jax import lax
from jax.experimental import pallas as pl
from jax.experimental.pallas import tpu as pltpu

N_DEV = 32
PLANE = 8
NZ = 4
SQ = 1024
DM = 1024
CHUNK = DM // PLANE
SUB = CHUNK // NZ
DH = 128

_MESH = pl.DeviceIdType.MESH


def _allreduce_body(p_ref, out_ref, comm_cw, comm_ccw, zcomm,
                    send_cw, recv_cw, send_ccw, recv_ccw,
                    send_b1, recv_b1, send_b2, recv_b2,
                    send_p3cw, recv_p3cw, send_p3ccw, recv_p3ccw, p1done):
    my = lax.axis_index("i")
    q = lax.rem(my, PLANE)
    zi = my // PLANE
    base = my - q
    pright = base + lax.rem(q + 1, PLANE)
    pleft = base + lax.rem(q + PLANE - 1, PLANE)

    barrier_sem = pltpu.get_barrier_semaphore()
    peers = [pleft, pright] + [
        lax.rem(zi + d, NZ) * PLANE + q for d in range(1, NZ)
    ]
    for pr in peers:
        pl.semaphore_signal(barrier_sem, inc=1,
                            device_id=(pr,), device_id_type=_MESH)
    pl.semaphore_wait(barrier_sem, len(peers))

    out_ref[...] = p_ref[...]

    for s in range(4):
        rdma_cw = pltpu.make_async_remote_copy(
            src_ref=out_ref.at[pl.ds(lax.rem(q + 5 - s + PLANE, PLANE) * CHUNK, CHUNK), :],
            dst_ref=comm_cw.at[s],
            send_sem=send_cw,
            recv_sem=recv_cw.at[s],
            device_id=(pright,),
            device_id_type=_MESH,
        )
        rdma_cw.start()
        if s < 3:
            rdma_ccw = pltpu.make_async_remote_copy(
                src_ref=out_ref.at[pl.ds(lax.rem(q + 6 + s, PLANE) * CHUNK, CHUNK), :],
                dst_ref=comm_ccw.at[s],
                send_sem=send_ccw,
                recv_sem=recv_ccw.at[s],
                device_id=(pleft,),
                device_id_type=_MESH,
            )
            rdma_ccw.start()
        rdma_cw.wait()
        rows = pl.ds(lax.rem(q + 4 - s + PLANE, PLANE) * CHUNK, CHUNK)
        out_ref[rows, :] = out_ref[rows, :] + comm_cw[s]
        if s < 3:
            rdma_ccw.wait()
            rows = pl.ds(lax.rem(q + 7 + s, PLANE) * CHUNK, CHUNK)
            out_ref[rows, :] = out_ref[rows, :] + comm_ccw[s]

    for pr in (pleft, pright):
        pl.semaphore_signal(p1done, inc=1, device_id=(pr,),
                            device_id_type=_MESH)

    c_own = lax.rem(q + 1, PLANE)
    slab = c_own * CHUNK

    b1 = []
    for d in range(1, NZ):
        tz = lax.rem(zi + d, NZ)
        tgt = tz * PLANE + q
        rdma = pltpu.make_async_remote_copy(
            src_ref=out_ref.at[pl.ds(slab + tz * SUB, SUB), :],
            dst_ref=zcomm.at[zi],
            send_sem=send_b1.at[d - 1],
            recv_sem=recv_b1.at[zi],
            device_id=(tgt,),
            device_id_type=_MESH,
        )
        rdma.start()
        b1.append(rdma)
    for d in range(1, NZ):
        sz = lax.rem(zi + d, NZ)
        rw = pltpu.make_async_remote_copy(
            src_ref=zcomm.at[sz],
            dst_ref=zcomm.at[sz],
            send_sem=send_b1.at[d - 1],
            recv_sem=recv_b1.at[sz],
            device_id=(my,),
            device_id_type=_MESH,
        )
        rw.wait_recv()
    myrows = pl.ds(slab + zi * SUB, SUB)
    acc = out_ref[myrows, :]
    for d in range(1, NZ):
        sz = lax.rem(zi + d, NZ)
        acc = acc + zcomm[sz]
    out_ref[myrows, :] = acc
    for rdma in b1:
        rdma.wait_send()

    b2 = []
    for d in range(1, NZ):
        tz = lax.rem(zi + d, NZ)
        tgt = tz * PLANE + q
        rdma = pltpu.make_async_remote_copy(
            src_ref=out_ref.at[myrows, :],
            dst_ref=out_ref.at[myrows, :],
            send_sem=send_b2.at[d - 1],
            recv_sem=recv_b2.at[zi],
            device_id=(tgt,),
            device_id_type=_MESH,
        )
        rdma.start()
        b2.append(rdma)
    for d in range(1, NZ):
        sz = lax.rem(zi + d, NZ)
        rw = pltpu.make_async_remote_copy(
            src_ref=out_ref.at[pl.ds(slab + sz * SUB, SUB), :],
            dst_ref=out_ref.at[pl.ds(slab + sz * SUB, SUB), :],
            send_sem=send_b2.at[d - 1],
            recv_sem=recv_b2.at[sz],
            device_id=(my,),
            device_id_type=_MESH,
        )
        rw.wait_recv()
    for rdma in b2:
        rdma.wait_send()

    pl.semaphore_wait(p1done, 2)
    for s in range(4):
        rows_cw = pl.ds(lax.rem(q + PLANE + 1 - s, PLANE) * CHUNK, CHUNK)
        rdma_cw = pltpu.make_async_remote_copy(
            src_ref=out_ref.at[rows_cw, :],
            dst_ref=out_ref.at[rows_cw, :],
            send_sem=send_p3cw,
            recv_sem=recv_p3cw.at[s],
            device_id=(pright,),
            device_id_type=_MESH,
        )
        rdma_cw.start()
        if s < 3:
            rows_ccw = pl.ds(lax.rem(q + 1 + s, PLANE) * CHUNK, CHUNK)
            rdma_ccw = pltpu.make_async_remote_copy(
                src_ref=out_ref.at[rows_ccw, :],
                dst_ref=out_ref.at[rows_ccw, :],
                send_sem=send_p3ccw,
                recv_sem=recv_p3ccw.at[s],
                device_id=(pleft,),
                device_id_type=_MESH,
            )
            rdma_ccw.start()
        rdma_cw.wait()
        if s < 3:
            rdma_ccw.wait()


def _ring_allreduce(partial):
    return pl.pallas_call(
        _allreduce_body,
        out_shape=jax.ShapeDtypeStruct((DM, DM), jnp.float32),
        in_specs=[pl.BlockSpec(memory_space=pltpu.VMEM)],
        out_specs=pl.BlockSpec(memory_space=pltpu.VMEM),
        scratch_shapes=[
            pltpu.VMEM((4, CHUNK, DM), jnp.float32),
            pltpu.VMEM((3, CHUNK, DM), jnp.float32),
            pltpu.VMEM((NZ, SUB, DM), jnp.float32),
            pltpu.SemaphoreType.DMA,
            pltpu.SemaphoreType.DMA((4,)),
            pltpu.SemaphoreType.DMA,
            pltpu.SemaphoreType.DMA((3,)),
            pltpu.SemaphoreType.DMA((NZ - 1,)),
            pltpu.SemaphoreType.DMA((NZ,)),
            pltpu.SemaphoreType.DMA((NZ - 1,)),
            pltpu.SemaphoreType.DMA((NZ,)),
            pltpu.SemaphoreType.DMA,
            pltpu.SemaphoreType.DMA((4,)),
            pltpu.SemaphoreType.DMA,
            pltpu.SemaphoreType.DMA((3,)),
            pltpu.SemaphoreType.REGULAR,
        ],
        compiler_params=pltpu.CompilerParams(collective_id=0),
    )(partial)


def kernel(x, Wq, K_ext, V_ext, Wo):
    i = lax.axis_index("i")
    hs = Wq.shape[1] // DH

    xb = x[0].astype(jnp.bfloat16)
    q = jnp.dot(xb, Wq.astype(jnp.bfloat16),
                preferred_element_type=jnp.float32)
    q = q.reshape(SQ, hs, DH).astype(jnp.bfloat16)

    k = lax.dynamic_slice_in_dim(K_ext[0], i * hs, hs, axis=1)
    v = lax.dynamic_slice_in_dim(V_ext[0], i * hs, hs, axis=1)
    k = k.astype(jnp.bfloat16)
    v = v.astype(jnp.bfloat16)

    scores = jnp.einsum("ihd,jhd->hij", q, k,
                        preferred_element_type=jnp.float32)
    scores = scores * 0.08838834764831843

    qi = jnp.arange(SQ)[:, None]
    ki = jnp.arange(SQ)[None, :]
    mask = (jnp.abs(qi - ki) <= 128) | (ki < 32) | (qi < 32)
    scores = jnp.where(mask[None], scores, -1e9)

    m = scores.max(axis=-1, keepdims=True)
    w = jnp.exp(scores - m)
    w = w / w.sum(axis=-1, keepdims=True)

    ctx = jnp.einsum("hij,jhd->ihd", w.astype(jnp.bfloat16), v,
                     preferred_element_type=jnp.float32)
    ctx = ctx.reshape(SQ, hs * DH).astype(jnp.bfloat16)

    partial = jnp.dot(ctx, Wo.astype(jnp.bfloat16),
                      preferred_element_type=jnp.float32)

    out = _ring_allreduce(partial)
    return out.reshape(1, SQ, DM)


# device time: 115576 ns/iter; 2.2324x vs baseline; 1.4425x over previous
import jax
import jax.numpy as jnp
from jax import lax
from jax.experimental import pallas as pl
from jax.experimental.pallas import tpu as pltpu

N_DEV = 32
PLANE = 8
NZ = 4
SQ = 1024
DM = 1024
CHUNK = DM // PLANE
SUB = CHUNK // NZ
DH = 128

_MESH = pl.DeviceIdType.MESH
_BF = jnp.bfloat16
_F32 = jnp.float32


def _allreduce_body(p_ref, out_ref,
                    comm_cw, comm_ccw, stage_cw, stage_ccw,
                    zcomm, zbcast, stage_b1, stage_b2,
                    p3cw, p3ccw,
                    send_cw, recv_cw, send_ccw, recv_ccw,
                    send_b1, recv_b1, send_b2, recv_b2,
                    send_p3cw, recv_p3cw, send_p3ccw, recv_p3ccw):
    my = lax.axis_index("i")
    q = lax.rem(my, PLANE)
    zi = my // PLANE
    base = my - q
    pright = base + lax.rem(q + 1, PLANE)
    pleft = base + lax.rem(q + PLANE - 1, PLANE)

    barrier_sem = pltpu.get_barrier_semaphore()
    peers = [pleft, pright] + [
        lax.rem(zi + d, NZ) * PLANE + q for d in range(1, NZ)
    ]
    for pr in peers:
        pl.semaphore_signal(barrier_sem, inc=1,
                            device_id=(pr,), device_id_type=_MESH)
    pl.semaphore_wait(barrier_sem, len(peers))

    out_ref[...] = p_ref[...]

    for s in range(4):
        rows_s = pl.ds(lax.rem(q + 5 - s + PLANE, PLANE) * CHUNK, CHUNK)
        stage_cw[...] = out_ref[rows_s, :].astype(_BF)
        rdma_cw = pltpu.make_async_remote_copy(
            src_ref=stage_cw,
            dst_ref=comm_cw.at[s],
            send_sem=send_cw,
            recv_sem=recv_cw.at[s],
            device_id=(pright,),
            device_id_type=_MESH,
        )
        rdma_cw.start()
        if s < 3:
            rows_t = pl.ds(lax.rem(q + 6 + s, PLANE) * CHUNK, CHUNK)
            stage_ccw[...] = out_ref[rows_t, :].astype(_BF)
            rdma_ccw = pltpu.make_async_remote_copy(
                src_ref=stage_ccw,
                dst_ref=comm_ccw.at[s],
                send_sem=send_ccw,
                recv_sem=recv_ccw.at[s],
                device_id=(pleft,),
                device_id_type=_MESH,
            )
            rdma_ccw.start()
        rdma_cw.wait()
        rows = pl.ds(lax.rem(q + 4 - s + PLANE, PLANE) * CHUNK, CHUNK)
        out_ref[rows, :] = out_ref[rows, :] + comm_cw[s].astype(_F32)
        if s < 3:
            rdma_ccw.wait()
            rows = pl.ds(lax.rem(q + 7 + s, PLANE) * CHUNK, CHUNK)
            out_ref[rows, :] = out_ref[rows, :] + comm_ccw[s].astype(_F32)

    c_own = lax.rem(q + 1, PLANE)
    slab = c_own * CHUNK

    b1 = []
    for d in range(1, NZ):
        tz = lax.rem(zi + d, NZ)
        tgt = tz * PLANE + q
        stage_b1[d - 1, :, :] = out_ref[pl.ds(slab + tz * SUB, SUB), :].astype(_BF)
        rdma = pltpu.make_async_remote_copy(
            src_ref=stage_b1.at[d - 1],
            dst_ref=zcomm.at[zi],
            send_sem=send_b1.at[d - 1],
            recv_sem=recv_b1.at[zi],
            device_id=(tgt,),
            device_id_type=_MESH,
        )
        rdma.start()
        b1.append(rdma)
    for d in range(1, NZ):
        sz = lax.rem(zi + d, NZ)
        rw = pltpu.make_async_remote_copy(
            src_ref=zcomm.at[sz],
            dst_ref=zcomm.at[sz],
            send_sem=send_b1.at[d - 1],
            recv_sem=recv_b1.at[sz],
            device_id=(my,),
            device_id_type=_MESH,
        )
        rw.wait_recv()
    myrows = pl.ds(slab + zi * SUB, SUB)
    acc = out_ref[myrows, :]
    for d in range(1, NZ):
        sz = lax.rem(zi + d, NZ)
        acc = acc + zcomm[sz].astype(_F32)
    out_ref[myrows, :] = acc
    stage_b2[...] = acc.astype(_BF)
    for rdma in b1:
        rdma.wait_send()

    b2 = []
    for d in range(1, NZ):
        tz = lax.rem(zi + d, NZ)
        tgt = tz * PLANE + q
        rdma = pltpu.make_async_remote_copy(
            src_ref=stage_b2,
            dst_ref=zbcast.at[zi],
            send_sem=send_b2.at[d - 1],
            recv_sem=recv_b2.at[zi],
            device_id=(tgt,),
            device_id_type=_MESH,
        )
        rdma.start()
        b2.append(rdma)
    for d in range(1, NZ):
        sz = lax.rem(zi + d, NZ)
        rw = pltpu.make_async_remote_copy(
            src_ref=zbcast.at[sz],
            dst_ref=zbcast.at[sz],
            send_sem=send_b2.at[d - 1],
            recv_sem=recv_b2.at[sz],
            device_id=(my,),
            device_id_type=_MESH,
        )
        rw.wait_recv()
        out_ref[pl.ds(slab + sz * SUB, SUB), :] = zbcast[sz].astype(_F32)
    for rdma in b2:
        rdma.wait_send()

    stage_cw[...] = out_ref[pl.ds(slab, CHUNK), :].astype(_BF)
    for s in range(4):
        rdma_cw = pltpu.make_async_remote_copy(
            src_ref=stage_cw if s == 0 else p3cw.at[s - 1],
            dst_ref=p3cw.at[s],
            send_sem=send_p3cw,
            recv_sem=recv_p3cw.at[s],
            device_id=(pright,),
            device_id_type=_MESH,
        )
        rdma_cw.start()
        if s < 3:
            rdma_ccw = pltpu.make_async_remote_copy(
                src_ref=stage_cw if s == 0 else p3ccw.at[s - 1],
                dst_ref=p3ccw.at[s],
                send_sem=send_p3ccw,
                recv_sem=recv_p3ccw.at[s],
                device_id=(pleft,),
                device_id_type=_MESH,
            )
            rdma_ccw.start()
        rdma_cw.wait()
        rows = pl.ds(lax.rem(q + PLANE - s, PLANE) * CHUNK, CHUNK)
        out_ref[rows, :] = p3cw[s].astype(_F32)
        if s < 3:
            rdma_ccw.wait()
            rows = pl.ds(lax.rem(q + 2 + s, PLANE) * CHUNK, CHUNK)
            out_ref[rows, :] = p3ccw[s].astype(_F32)


def _ring_allreduce(partial):
    return pl.pallas_call(
        _allreduce_body,
        out_shape=jax.ShapeDtypeStruct((DM, DM), jnp.float32),
        in_specs=[pl.BlockSpec(memory_space=pltpu.VMEM)],
        out_specs=pl.BlockSpec(memory_space=pltpu.VMEM),
        scratch_shapes=[
            pltpu.VMEM((4, CHUNK, DM), _BF),
            pltpu.VMEM((3, CHUNK, DM), _BF),
            pltpu.VMEM((CHUNK, DM), _BF),
            pltpu.VMEM((CHUNK, DM), _BF),
            pltpu.VMEM((NZ, SUB, DM), _BF),
            pltpu.VMEM((NZ, SUB, DM), _BF),
            pltpu.VMEM((NZ - 1, SUB, DM), _BF),
            pltpu.VMEM((SUB, DM), _BF),
            pltpu.VMEM((4, CHUNK, DM), _BF),
            pltpu.VMEM((3, CHUNK, DM), _BF),
            pltpu.SemaphoreType.DMA,
            pltpu.SemaphoreType.DMA((4,)),
            pltpu.SemaphoreType.DMA,
            pltpu.SemaphoreType.DMA((3,)),
            pltpu.SemaphoreType.DMA((NZ - 1,)),
            pltpu.SemaphoreType.DMA((NZ,)),
            pltpu.SemaphoreType.DMA((NZ - 1,)),
            pltpu.SemaphoreType.DMA((NZ,)),
            pltpu.SemaphoreType.DMA,
            pltpu.SemaphoreType.DMA((4,)),
            pltpu.SemaphoreType.DMA,
            pltpu.SemaphoreType.DMA((3,)),
        ],
        compiler_params=pltpu.CompilerParams(collective_id=0),
    )(partial)


def kernel(x, Wq, K_ext, V_ext, Wo):
    i = lax.axis_index("i")
    hs = Wq.shape[1] // DH

    xb = x[0].astype(jnp.bfloat16)
    q = jnp.dot(xb, Wq.astype(jnp.bfloat16),
                preferred_element_type=jnp.float32)
    q = q.reshape(SQ, hs, DH).astype(jnp.bfloat16)

    k = lax.dynamic_slice_in_dim(K_ext[0], i * hs, hs, axis=1)
    v = lax.dynamic_slice_in_dim(V_ext[0], i * hs, hs, axis=1)
    k = k.astype(jnp.bfloat16)
    v = v.astype(jnp.bfloat16)

    scores = jnp.einsum("ihd,jhd->hij", q, k,
                        preferred_element_type=jnp.float32)
    scores = scores * 0.08838834764831843

    qi = jnp.arange(SQ)[:, None]
    ki = jnp.arange(SQ)[None, :]
    mask = (jnp.abs(qi - ki) <= 128) | (ki < 32) | (qi < 32)
    scores = jnp.where(mask[None], scores, -1e9)

    m = scores.max(axis=-1, keepdims=True)
    w = jnp.exp(scores - m)
    w = w / w.sum(axis=-1, keepdims=True)

    ctx = jnp.einsum("hij,jhd->ihd", w.astype(jnp.bfloat16), v,
                     preferred_element_type=jnp.float32)
    ctx = ctx.reshape(SQ, hs * DH).astype(jnp.bfloat16)

    partial = jnp.dot(ctx, Wo.astype(jnp.bfloat16),
                      preferred_element_type=jnp.float32)

    out = _ring_allreduce(partial)
    return out.reshape(1, SQ, DM)


# device time: 111303 ns/iter; 2.3181x vs baseline; 1.0384x over previous
import jax
import jax.numpy as jnp
from jax import lax
from jax.experimental import pallas as pl
from jax.experimental.pallas import tpu as pltpu

N_DEV = 32
PLANE = 8
NZ = 4
SQ = 1024
DM = 1024
CHUNK = DM // PLANE
SUB = CHUNK // NZ
DH = 128

_MESH = pl.DeviceIdType.MESH
_BF = jnp.bfloat16
_F32 = jnp.float32


def _allreduce_body(p_ref, out_ref,
                    comm_cw, comm_ccw, stage_cw, stage_ccw,
                    zcomm, zbcast, stage_b1, stage_b2,
                    p3cw, p3ccw,
                    send_cw, recv_cw, send_ccw, recv_ccw,
                    send_b1, recv_b1, send_b2, recv_b2,
                    send_p3cw, recv_p3cw, send_p3ccw, recv_p3ccw):
    my = lax.axis_index("i")
    q = lax.rem(my, PLANE)
    zi = my // PLANE
    base = my - q
    pright = base + lax.rem(q + 1, PLANE)
    pleft = base + lax.rem(q + PLANE - 1, PLANE)

    barrier_sem = pltpu.get_barrier_semaphore()
    peers = [pleft, pright] + [
        lax.rem(zi + d, NZ) * PLANE + q for d in range(1, NZ)
    ]
    for pr in peers:
        pl.semaphore_signal(barrier_sem, inc=1,
                            device_id=(pr,), device_id_type=_MESH)
    pl.semaphore_wait(barrier_sem, len(peers))

    out_ref[...] = p_ref[...]

    for s in range(4):
        rows_s = pl.ds(lax.rem(q + 5 - s + PLANE, PLANE) * CHUNK, CHUNK)
        stage_cw[...] = out_ref[rows_s, :].astype(_BF)
        rdma_cw = pltpu.make_async_remote_copy(
            src_ref=stage_cw,
            dst_ref=comm_cw.at[s],
            send_sem=send_cw,
            recv_sem=recv_cw.at[s],
            device_id=(pright,),
            device_id_type=_MESH,
        )
        rdma_cw.start()
        if s < 3:
            rows_t = pl.ds(lax.rem(q + 6 + s, PLANE) * CHUNK, CHUNK)
            stage_ccw[...] = out_ref[rows_t, :].astype(_BF)
            rdma_ccw = pltpu.make_async_remote_copy(
                src_ref=stage_ccw,
                dst_ref=comm_ccw.at[s],
                send_sem=send_ccw,
                recv_sem=recv_ccw.at[s],
                device_id=(pleft,),
                device_id_type=_MESH,
            )
            rdma_ccw.start()
        rdma_cw.wait()
        rows = pl.ds(lax.rem(q + 4 - s + PLANE, PLANE) * CHUNK, CHUNK)
        out_ref[rows, :] = out_ref[rows, :] + comm_cw[s].astype(_F32)
        if s < 3:
            rdma_ccw.wait()
            rows = pl.ds(lax.rem(q + 7 + s, PLANE) * CHUNK, CHUNK)
            out_ref[rows, :] = out_ref[rows, :] + comm_ccw[s].astype(_F32)

    c_own = lax.rem(q + 1, PLANE)
    slab = c_own * CHUNK

    b1 = []
    for d in range(1, NZ):
        tz = lax.rem(zi + d, NZ)
        tgt = tz * PLANE + q
        stage_b1[d - 1, :, :] = out_ref[pl.ds(slab + tz * SUB, SUB), :].astype(_BF)
        rdma = pltpu.make_async_remote_copy(
            src_ref=stage_b1.at[d - 1],
            dst_ref=zcomm.at[zi],
            send_sem=send_b1.at[d - 1],
            recv_sem=recv_b1.at[zi],
            device_id=(tgt,),
            device_id_type=_MESH,
        )
        rdma.start()
        b1.append(rdma)
    for d in range(1, NZ):
        sz = lax.rem(zi + d, NZ)
        rw = pltpu.make_async_remote_copy(
            src_ref=zcomm.at[sz],
            dst_ref=zcomm.at[sz],
            send_sem=send_b1.at[d - 1],
            recv_sem=recv_b1.at[sz],
            device_id=(my,),
            device_id_type=_MESH,
        )
        rw.wait_recv()
    myrows = pl.ds(slab + zi * SUB, SUB)
    acc = out_ref[myrows, :]
    for d in range(1, NZ):
        sz = lax.rem(zi + d, NZ)
        acc = acc + zcomm[sz].astype(_F32)
    out_ref[myrows, :] = acc
    stage_b2[...] = acc.astype(_BF)
    for rdma in b1:
        rdma.wait_send()

    b2 = []
    for d in range(1, NZ):
        tz = lax.rem(zi + d, NZ)
        tgt = tz * PLANE + q
        rdma = pltpu.make_async_remote_copy(
            src_ref=stage_b2,
            dst_ref=zbcast.at[zi],
            send_sem=send_b2.at[d - 1],
            recv_sem=recv_b2.at[zi],
            device_id=(tgt,),
            device_id_type=_MESH,
        )
        rdma.start()
        b2.append(rdma)
    for d in range(1, NZ):
        sz = lax.rem(zi + d, NZ)
        rw = pltpu.make_async_remote_copy(
            src_ref=zbcast.at[sz],
            dst_ref=zbcast.at[sz],
            send_sem=send_b2.at[d - 1],
            recv_sem=recv_b2.at[sz],
            device_id=(my,),
            device_id_type=_MESH,
        )
        rw.wait_recv()
        out_ref[pl.ds(slab + sz * SUB, SUB), :] = zbcast[sz].astype(_F32)
    for rdma in b2:
        rdma.wait_send()

    stage_cw[...] = out_ref[pl.ds(slab, CHUNK), :].astype(_BF)
    for s in range(4):
        rdma_cw = pltpu.make_async_remote_copy(
            src_ref=stage_cw if s == 0 else p3cw.at[s - 1],
            dst_ref=p3cw.at[s],
            send_sem=send_p3cw,
            recv_sem=recv_p3cw.at[s],
            device_id=(pright,),
            device_id_type=_MESH,
        )
        rdma_cw.start()
        if s < 3:
            rdma_ccw = pltpu.make_async_remote_copy(
                src_ref=stage_cw if s == 0 else p3ccw.at[s - 1],
                dst_ref=p3ccw.at[s],
                send_sem=send_p3ccw,
                recv_sem=recv_p3ccw.at[s],
                device_id=(pleft,),
                device_id_type=_MESH,
            )
            rdma_ccw.start()
        rdma_cw.wait()
        rows = pl.ds(lax.rem(q + PLANE - s, PLANE) * CHUNK, CHUNK)
        out_ref[rows, :] = p3cw[s].astype(_F32)
        if s < 3:
            rdma_ccw.wait()
            rows = pl.ds(lax.rem(q + 2 + s, PLANE) * CHUNK, CHUNK)
            out_ref[rows, :] = p3ccw[s].astype(_F32)


def _ring_allreduce(partial):
    return pl.pallas_call(
        _allreduce_body,
        out_shape=jax.ShapeDtypeStruct((DM, DM), jnp.float32),
        in_specs=[pl.BlockSpec(memory_space=pltpu.VMEM)],
        out_specs=pl.BlockSpec(memory_space=pltpu.VMEM),
        scratch_shapes=[
            pltpu.VMEM((4, CHUNK, DM), _BF),
            pltpu.VMEM((3, CHUNK, DM), _BF),
            pltpu.VMEM((CHUNK, DM), _BF),
            pltpu.VMEM((CHUNK, DM), _BF),
            pltpu.VMEM((NZ, SUB, DM), _BF),
            pltpu.VMEM((NZ, SUB, DM), _BF),
            pltpu.VMEM((NZ - 1, SUB, DM), _BF),
            pltpu.VMEM((SUB, DM), _BF),
            pltpu.VMEM((4, CHUNK, DM), _BF),
            pltpu.VMEM((3, CHUNK, DM), _BF),
            pltpu.SemaphoreType.DMA,
            pltpu.SemaphoreType.DMA((4,)),
            pltpu.SemaphoreType.DMA,
            pltpu.SemaphoreType.DMA((3,)),
            pltpu.SemaphoreType.DMA((NZ - 1,)),
            pltpu.SemaphoreType.DMA((NZ,)),
            pltpu.SemaphoreType.DMA((NZ - 1,)),
            pltpu.SemaphoreType.DMA((NZ,)),
            pltpu.SemaphoreType.DMA,
            pltpu.SemaphoreType.DMA((4,)),
            pltpu.SemaphoreType.DMA,
            pltpu.SemaphoreType.DMA((3,)),
        ],
        compiler_params=pltpu.CompilerParams(collective_id=0),
    )(partial)


def kernel(x, Wq, K_ext, V_ext, Wo):
    i = lax.axis_index("i")
    hs = Wq.shape[1] // DH
    scale = 0.08838834764831843

    xb = x[0].astype(jnp.bfloat16)
    qall = jnp.dot(xb, Wq.astype(jnp.bfloat16),
                   preferred_element_type=jnp.float32)
    qall = qall.reshape(SQ, hs, DH).astype(jnp.bfloat16)

    k = lax.dynamic_slice_in_dim(K_ext[0], i * hs, hs, axis=1)
    v = lax.dynamic_slice_in_dim(V_ext[0], i * hs, hs, axis=1)
    k = k.astype(jnp.bfloat16)
    v = v.astype(jnp.bfloat16)

    BQ, W = 256, 512
    ctx_blocks = []
    for b in range(SQ // BQ):
        k0 = min(max(BQ * b - 128, 0), SQ - W)
        qb = lax.slice_in_dim(qall, b * BQ, (b + 1) * BQ, axis=0)
        kw = lax.slice_in_dim(k, k0, k0 + W, axis=0)
        vw = lax.slice_in_dim(v, k0, k0 + W, axis=0)
        sw = jnp.einsum("ihd,jhd->hij", qb, kw,
                        preferred_element_type=jnp.float32) * scale
        qi = (b * BQ + jnp.arange(BQ))[:, None]
        ki = (k0 + jnp.arange(W))[None, :]
        live = (jnp.abs(qi - ki) <= 128) | (ki < 32)
        sw = jnp.where(live[None], sw, -1e9)
        if b == 0:
            s, vv = sw, vw
        else:
            sg = jnp.einsum("ihd,jhd->hij", qb, k[:32],
                            preferred_element_type=jnp.float32) * scale
            s = jnp.concatenate([sg, sw], axis=-1)
            vv = jnp.concatenate([v[:32], vw], axis=0)
        wts = jnp.exp(s)
        wts = wts / wts.sum(axis=-1, keepdims=True)
        ctx_blocks.append(
            jnp.einsum("hij,jhd->ihd", wts.astype(jnp.bfloat16), vv,
                       preferred_element_type=jnp.float32))

    s0 = jnp.einsum("ihd,jhd->hij", qall[:32], k,
                    preferred_element_type=jnp.float32) * scale
    w0 = jnp.exp(s0)
    w0 = w0 / w0.sum(axis=-1, keepdims=True)
    ctx0 = jnp.einsum("hij,jhd->ihd", w0.astype(jnp.bfloat16), v,
                      preferred_element_type=jnp.float32)

    ctx = jnp.concatenate([ctx0, ctx_blocks[0][32:]] + ctx_blocks[1:],
                          axis=0)
    ctx = ctx.reshape(SQ, hs * DH).astype(jnp.bfloat16)

    partial = jnp.dot(ctx, Wo.astype(jnp.bfloat16),
                      preferred_element_type=jnp.float32)

    out = _ring_allreduce(partial)
    return out.reshape(1, SQ, DM)
